# baseline (speedup 1.0000x reference)
import jax
import jax.numpy as jnp
from jax.experimental import pallas as pl


_N = 4608
_Q = 512
_E = 32
_R = 1152
_K = 3
_IMAX = 0x7FFFFFFF
_MASK16 = -0x10000


def _hi_part(a):
    u = jax.lax.bitcast_convert_type(a, jnp.int32)
    return jax.lax.bitcast_convert_type(u & jnp.int32(_MASK16), jnp.float32)


def _exact_dist(wrow, xb):
    d = wrow - xb
    sq = d * d
    blocks = []
    for g in range(4):
        b = sq[:, 8 * g:8 * g + 8]
        u = b[:, 0:4] + b[:, 4:8]
        v = u[:, 0:2] + u[:, 2:4]
        blocks.append(v[:, 0:1] + v[:, 1:2])
    return ((blocks[0] + blocks[1]) + blocks[2]) + blocks[3]


def _vq_body(x_ref, w_ref, qd_ref, qi_ref):
    xb = x_ref[...]
    w = w_ref[...]
    wt = w.T

    wn = jnp.sum(wt * wt, axis=0)[None, :]

    whi_f = _hi_part(w)
    r1 = w - whi_f
    wlo_f = _hi_part(r1)
    whi = whi_f.astype(jnp.bfloat16)
    wlo = wlo_f.astype(jnp.bfloat16)
    wll = (r1 - wlo_f).astype(jnp.bfloat16)

    wt2 = wt + wt
    w2h_f = _hi_part(wt2)
    w2h = w2h_f.astype(jnp.bfloat16)
    w2l = (wt2 - w2h_f).astype(jnp.bfloat16)

    xh_f = _hi_part(xb)
    xh = xh_f.astype(jnp.bfloat16)
    xl = (xb - xh_f).astype(jnp.bfloat16)
    xw2 = (
        jax.lax.dot(xh, w2h, preferred_element_type=jnp.float32)
        + jax.lax.dot(xh, w2l, preferred_element_type=jnp.float32)
        + jax.lax.dot(xl, w2h, preferred_element_type=jnp.float32)
    )
    s = wn - xw2

    u = jax.lax.bitcast_convert_type(s, jnp.int32)
    k = u ^ jax.lax.shift_right_logical(
        jax.lax.shift_right_arithmetic(u, 31), 1)
    qiota = jax.lax.broadcasted_iota(jnp.int32, (_R, _Q), 1)
    key = (k & jnp.int32(~511)) | qiota

    hits = []
    idxs = []
    for _ in range(_K):
        mk = jnp.min(key, axis=1, keepdims=True)
        hit = key == mk
        key = jnp.where(hit, _IMAX, key)
        hits.append(hit)
        idxs.append(mk[:, 0] & 511)

    onehot = jnp.concatenate(hits, axis=0).astype(jnp.bfloat16)
    rows = jax.lax.dot(onehot, wll, preferred_element_type=jnp.float32)
    for wpart in (wlo, whi):
        rows = rows + jax.lax.dot(onehot, wpart,
                                  preferred_element_type=jnp.float32)
    xb3 = jnp.concatenate([xb] * _K, axis=0)
    dall = _exact_dist(rows, xb3)[:, 0]

    best_d = dall[0:_R]
    best_i = idxs[0]
    best_row = rows[0:_R, :]
    for kk in range(1, _K):
        d = dall[kk * _R:(kk + 1) * _R]
        idx = idxs[kk]
        take = (d < best_d) | ((d == best_d) & (idx < best_i))
        best_d = jnp.where(take, d, best_d)
        best_i = jnp.where(take, idx, best_i)
        best_row = jnp.where(take[:, None], rows[kk * _R:(kk + 1) * _R, :],
                             best_row)

    qd_ref[...] = best_row
    qi_ref[0, 0, :] = best_i


@jax.jit
def _vq(xf, w):
    nb = _N // _R
    qd, qi = pl.pallas_call(
        _vq_body,
        grid=(nb,),
        in_specs=[
            pl.BlockSpec((_R, _E), lambda i: (i, 0)),
            pl.BlockSpec((_Q, _E), lambda i: (0, 0)),
        ],
        out_specs=[
            pl.BlockSpec((_R, _E), lambda i: (i, 0)),
            pl.BlockSpec((1, 1, _R), lambda i: (i, 0, 0)),
        ],
        out_shape=[
            jax.ShapeDtypeStruct((_N, _E), jnp.float32),
            jax.ShapeDtypeStruct((nb, 1, _R), jnp.int32),
        ],
    )(xf, w)
    return qd, qi


def kernel(x, weight):
    xf = x.reshape(_N, _E)
    qd, qi = _vq(xf, weight)
    return qd.reshape(x.shape), qi.reshape(x.shape[:-1])


# speedup vs baseline: 1.0072x; 1.0072x over previous
import jax
import jax.numpy as jnp
from jax.experimental import pallas as pl


_N = 4608
_Q = 512
_E = 32
_R = 1152
_K = 3
_IMAX = 0x7FFFFFFF
_MASK16 = -0x10000


def _hi_part(a):
    u = jax.lax.bitcast_convert_type(a, jnp.int32)
    return jax.lax.bitcast_convert_type(u & jnp.int32(_MASK16), jnp.float32)


def _exact_dist(wrow, xb):
    d = wrow - xb
    sq = d * d
    blocks = []
    for g in range(4):
        b = sq[:, 8 * g:8 * g + 8]
        u = b[:, 0:4] + b[:, 4:8]
        v = u[:, 0:2] + u[:, 2:4]
        blocks.append(v[:, 0:1] + v[:, 1:2])
    return ((blocks[0] + blocks[1]) + blocks[2]) + blocks[3]


def _vq_body(x_ref, wt_ref, w_ref, qd_ref, qi_ref):
    xb = x_ref[...]
    wt = wt_ref[...]
    w = w_ref[...]

    wn = jnp.sum(wt * wt, axis=0)[None, :]

    whi_f = _hi_part(w)
    r1 = w - whi_f
    wlo_f = _hi_part(r1)
    whi = whi_f.astype(jnp.bfloat16)
    wlo = wlo_f.astype(jnp.bfloat16)
    wll = (r1 - wlo_f).astype(jnp.bfloat16)

    wt2 = wt + wt
    w2h_f = _hi_part(wt2)
    w2h = w2h_f.astype(jnp.bfloat16)
    w2l = (wt2 - w2h_f).astype(jnp.bfloat16)

    xh_f = _hi_part(xb)
    xh = xh_f.astype(jnp.bfloat16)
    xl = (xb - xh_f).astype(jnp.bfloat16)
    xw2 = (
        jax.lax.dot(xh, w2h, preferred_element_type=jnp.float32)
        + jax.lax.dot(xh, w2l, preferred_element_type=jnp.float32)
        + jax.lax.dot(xl, w2h, preferred_element_type=jnp.float32)
    )
    s = wn - xw2

    u = jax.lax.bitcast_convert_type(s, jnp.int32)
    k = u ^ jax.lax.shift_right_logical(
        jax.lax.shift_right_arithmetic(u, 31), 1)
    qiota = jax.lax.broadcasted_iota(jnp.int32, (_R, _Q), 1)
    key = (k & jnp.int32(~511)) | qiota

    hits = []
    idxs = []
    for _ in range(_K):
        mk = jnp.min(key, axis=1, keepdims=True)
        hit = key == mk
        key = jnp.where(hit, _IMAX, key)
        hits.append(hit)
        idxs.append(mk[:, 0] & 511)

    onehot = jnp.concatenate(hits, axis=0).astype(jnp.bfloat16)
    rows = jax.lax.dot(onehot, wll, preferred_element_type=jnp.float32)
    for wpart in (wlo, whi):
        rows = rows + jax.lax.dot(onehot, wpart,
                                  preferred_element_type=jnp.float32)
    xb3 = jnp.concatenate([xb] * _K, axis=0)
    dall = _exact_dist(rows, xb3)[:, 0]

    best_d = dall[0:_R]
    best_i = idxs[0]
    best_row = rows[0:_R, :]
    for kk in range(1, _K):
        d = dall[kk * _R:(kk + 1) * _R]
        idx = idxs[kk]
        take = (d < best_d) | ((d == best_d) & (idx < best_i))
        best_d = jnp.where(take, d, best_d)
        best_i = jnp.where(take, idx, best_i)
        best_row = jnp.where(take[:, None], rows[kk * _R:(kk + 1) * _R, :],
                             best_row)

    qd_ref[...] = best_row
    qi_ref[0, 0, :] = best_i


@jax.jit
def _vq(xf, wt, w):
    nb = _N // _R
    qd, qi = pl.pallas_call(
        _vq_body,
        grid=(nb,),
        in_specs=[
            pl.BlockSpec((_R, _E), lambda i: (i, 0)),
            pl.BlockSpec((_E, _Q), lambda i: (0, 0)),
            pl.BlockSpec((_Q, _E), lambda i: (0, 0)),
        ],
        out_specs=[
            pl.BlockSpec((_R, _E), lambda i: (i, 0)),
            pl.BlockSpec((1, 1, _R), lambda i: (i, 0, 0)),
        ],
        out_shape=[
            jax.ShapeDtypeStruct((_N, _E), jnp.float32),
            jax.ShapeDtypeStruct((nb, 1, _R), jnp.int32),
        ],
    )(xf, wt, w)
    return qd, qi


def kernel(x, weight):
    xf = x.reshape(_N, _E)
    qd, qi = _vq(xf, weight.T, weight)
    return qd.reshape(x.shape), qi.reshape(x.shape[:-1])
